# baseline (speedup 1.0000x reference)
import functools

import jax
import jax.numpy as jnp
from jax import lax
from jax.experimental import pallas as pl
from jax.experimental.pallas import tpu as pltpu
from jax.experimental.pallas import tpu_sc as plsc

VOCAB = 1000000
N_EMBD = 1024
R = 128

NUM_CORES = 2
NUM_SUBCORES = 16
NW = NUM_CORES * NUM_SUBCORES

CHUNK = 80


def _make_gather(B):
  assert B % (8 * NW) == 0
  b_per_w = B // NW
  assert b_per_w % CHUNK == 0
  n_chunks = b_per_w // CHUNK
  mesh = plsc.VectorSubcoreMesh(core_axis_name="c", subcore_axis_name="s")

  @functools.partial(
      pl.kernel,
      out_type=jax.ShapeDtypeStruct((B, R), jnp.float32),
      mesh=mesh,
      compiler_params=pltpu.CompilerParams(use_tc_tiling_on_sc=True),
      scratch_types=[
          pltpu.VMEM((b_per_w,), jnp.int32),
          pltpu.VMEM((CHUNK, R), jnp.float32),
          pltpu.SemaphoreType.DMA,
      ],
  )
  def gather(table_hbm, idx_hbm, out_hbm, idx_v, rows_v, gsem):
    wid = lax.axis_index("s") * NUM_CORES + lax.axis_index("c")
    base = wid * b_per_w
    pltpu.sync_copy(idx_hbm.at[pl.ds(base, b_per_w)], idx_v)
    for c in range(n_chunks):
      pltpu.async_copy(table_hbm.at[idx_v.at[pl.ds(c * CHUNK, CHUNK)]],
                       rows_v, gsem).wait()
      pltpu.sync_copy(rows_v, out_hbm.at[pl.ds(base + c * CHUNK, CHUNK)])

  return gather


def _main_body(Lm, G, h_ref, w_ref, o_ref):
  y2 = lax.dot_general(
      h_ref[...], w_ref[...],
      dimension_numbers=(((1,), (1,)), ((), ())),
      preferred_element_type=jnp.float32)
  o_ref[...] = y2.reshape(G, Lm, N_EMBD)


def _tail_body(Lt, Lm, G, y_any, h_ref, w_ref, o_ref, y_v, sem):
  del y_any
  i = pl.program_id(0)
  y2 = lax.dot_general(
      h_ref[...], w_ref[...],
      dimension_numbers=(((1,), (1,)), ((), ())),
      preferred_element_type=jnp.float32)
  y_v[...] = y2.reshape(G, Lt, N_EMBD)
  pltpu.make_async_copy(
      y_v, o_ref.at[pl.ds(i * G, G), pl.ds(Lm, Lt), :], sem).start()
  pltpu.make_async_copy(
      y_v, o_ref.at[pl.ds(i * G, G), pl.ds(Lm, Lt), :], sem).wait()


def _project(h, w, Bo, L, G=64):
  Lm = (L // 8) * 8
  Lt = L - Lm
  assert Bo % G == 0 and (G * Lt) % 8 == 0
  out_shape = jax.ShapeDtypeStruct((Bo, L, N_EMBD), jnp.float32)

  y = pl.pallas_call(
      functools.partial(_main_body, Lm, G),
      grid=(Bo // G,),
      in_specs=[
          pl.BlockSpec((G * Lm, R), lambda i: (i, 0)),
          pl.BlockSpec((N_EMBD, R), lambda i: (0, 0)),
      ],
      out_specs=pl.BlockSpec((G, Lm, N_EMBD), lambda i: (i, 0, 0)),
      out_shape=out_shape,
  )(h, w)

  if Lt == 0:
    return y

  off_blocks = (Bo * Lm) // (G * Lt)
  assert (Bo * Lm) % (G * Lt) == 0
  y = pl.pallas_call(
      functools.partial(_tail_body, Lt, Lm, G),
      grid=(Bo // G,),
      in_specs=[
          pl.BlockSpec(memory_space=pl.ANY),
          pl.BlockSpec((G * Lt, R), lambda i: (i + off_blocks, 0)),
          pl.BlockSpec((N_EMBD, R), lambda i: (0, 0)),
      ],
      out_specs=pl.BlockSpec(memory_space=pl.ANY),
      out_shape=out_shape,
      scratch_shapes=[
          pltpu.VMEM((G, Lt, N_EMBD), jnp.float32),
          pltpu.SemaphoreType.DMA,
      ],
      input_output_aliases={0: 0},
  )(y, h, w)
  return y


def kernel(x, embed_in_weight, embed_out_weight):
  Bo, L = x.shape
  B = Bo * L
  Lm = (L // 8) * 8
  xi = x.astype(jnp.int32)
  xf = jnp.concatenate([xi[:, :Lm].reshape(-1), xi[:, Lm:].reshape(-1)])
  h = _make_gather(B)(embed_in_weight, xf)
  return _project(h, embed_out_weight, Bo, L)


# speedup vs baseline: 1.0238x; 1.0238x over previous
import functools

import jax
import jax.numpy as jnp
from jax import lax
from jax.experimental import pallas as pl
from jax.experimental.pallas import tpu as pltpu
from jax.experimental.pallas import tpu_sc as plsc

VOCAB = 1000000
N_EMBD = 1024
R = 128

NUM_CORES = 2
NUM_SUBCORES = 16
NW = NUM_CORES * NUM_SUBCORES

CHUNK = 80


def _make_gather(B):
  assert B % (8 * NW) == 0
  b_per_w = B // NW
  assert b_per_w % CHUNK == 0
  n_chunks = b_per_w // CHUNK
  mesh = plsc.VectorSubcoreMesh(core_axis_name="c", subcore_axis_name="s")

  @functools.partial(
      pl.kernel,
      out_type=jax.ShapeDtypeStruct((B, R), jnp.float32),
      mesh=mesh,
      compiler_params=pltpu.CompilerParams(use_tc_tiling_on_sc=True),
      scratch_types=[
          pltpu.VMEM((b_per_w,), jnp.int32),
          pltpu.VMEM((CHUNK, R), jnp.float32),
          pltpu.SemaphoreType.DMA,
      ],
  )
  def gather(table_hbm, idx_hbm, out_hbm, idx_v, rows_v, gsem):
    wid = lax.axis_index("s") * NUM_CORES + lax.axis_index("c")
    base = wid * b_per_w
    pltpu.sync_copy(idx_hbm.at[pl.ds(base, b_per_w)], idx_v)
    for c in range(n_chunks):
      pltpu.async_copy(table_hbm.at[idx_v.at[pl.ds(c * CHUNK, CHUNK)]],
                       rows_v, gsem).wait()
      pltpu.sync_copy(rows_v, out_hbm.at[pl.ds(base + c * CHUNK, CHUNK)])

  return gather


def _proj_body(L, G, h_ref, w_ref, o_ref):
  y2 = lax.dot_general(
      h_ref[...], w_ref[...],
      dimension_numbers=(((1,), (1,)), ((), ())),
      preferred_element_type=jnp.float32)
  o_ref[...] = y2.reshape(G, L, N_EMBD)


def _proj_body_alias(L, G, y_any, h_ref, w_ref, o_ref):
  del y_any
  _proj_body(L, G, h_ref, w_ref, o_ref)


def _project_phase(y_prev, h, w, Bo, L, seq0, nseq, G):
  assert nseq % G == 0 and seq0 % G == 0
  out_shape = jax.ShapeDtypeStruct((Bo, L, N_EMBD), jnp.float32)
  p0 = seq0 // G
  common = dict(
      grid=(nseq // G,),
      out_specs=pl.BlockSpec((G, L, N_EMBD), lambda i: (i + p0, 0, 0)),
      out_shape=out_shape,
  )
  h_spec = pl.BlockSpec((G * L, R), lambda i: (i, 0))
  w_spec = pl.BlockSpec((N_EMBD, R), lambda i: (0, 0))
  if y_prev is None:
    return pl.pallas_call(
        functools.partial(_proj_body, L, G),
        in_specs=[h_spec, w_spec],
        **common,
    )(h, w)
  return pl.pallas_call(
      functools.partial(_proj_body_alias, L, G),
      in_specs=[pl.BlockSpec(memory_space=pl.ANY), h_spec, w_spec],
      input_output_aliases={0: 0},
      **common,
  )(y_prev, h, w)


N_PHASES = 4


def kernel(x, embed_in_weight, embed_out_weight):
  Bo, L = x.shape
  xi = x.astype(jnp.int32)
  nseq = Bo // N_PHASES
  gather = _make_gather(nseq * L)
  hs = [gather(embed_in_weight, xi[p * nseq:(p + 1) * nseq].reshape(-1))
        for p in range(N_PHASES)]
  y = None
  for p in range(N_PHASES):
    y = _project_phase(y, hs[p], embed_out_weight, Bo, L,
                       seq0=p * nseq, nseq=nseq, G=64)
  return y


# speedup vs baseline: 1.0254x; 1.0015x over previous
import functools

import jax
import jax.numpy as jnp
from jax import lax
from jax.experimental import pallas as pl
from jax.experimental.pallas import tpu as pltpu
from jax.experimental.pallas import tpu_sc as plsc

VOCAB = 1000000
N_EMBD = 1024
R = 128

NUM_CORES = 2
NUM_SUBCORES = 16
NW = NUM_CORES * NUM_SUBCORES

CHUNK = 80


def _make_gather(B):
  assert B % (8 * NW) == 0
  b_per_w = B // NW
  assert b_per_w % CHUNK == 0
  n_chunks = b_per_w // CHUNK
  mesh = plsc.VectorSubcoreMesh(core_axis_name="c", subcore_axis_name="s")

  @functools.partial(
      pl.kernel,
      out_type=jax.ShapeDtypeStruct((B, R), jnp.float32),
      mesh=mesh,
      compiler_params=pltpu.CompilerParams(use_tc_tiling_on_sc=True),
      scratch_types=[
          pltpu.VMEM((b_per_w,), jnp.int32),
          pltpu.VMEM((CHUNK, R), jnp.float32),
          pltpu.SemaphoreType.DMA,
      ],
  )
  def gather(table_hbm, idx_hbm, out_hbm, idx_v, rows_v, gsem):
    wid = lax.axis_index("s") * NUM_CORES + lax.axis_index("c")
    base = wid * b_per_w
    pltpu.sync_copy(idx_hbm.at[pl.ds(base, b_per_w)], idx_v)
    for c in range(n_chunks):
      pltpu.async_copy(table_hbm.at[idx_v.at[pl.ds(c * CHUNK, CHUNK)]],
                       rows_v, gsem).wait()
      pltpu.sync_copy(rows_v, out_hbm.at[pl.ds(base + c * CHUNK, CHUNK)])

  return gather


def _proj_body(L, G, h_ref, w_ref, o_ref):
  y2 = lax.dot_general(
      h_ref[...], w_ref[...],
      dimension_numbers=(((1,), (1,)), ((), ())),
      preferred_element_type=jnp.float32)
  o_ref[...] = y2.reshape(G, L, N_EMBD)


def _proj_body_alias(L, G, y_any, h_ref, w_ref, o_ref):
  del y_any
  _proj_body(L, G, h_ref, w_ref, o_ref)


def _project_phase(y_prev, h, w, Bo, L, seq0, nseq, G):
  assert nseq % G == 0 and seq0 % G == 0
  out_shape = jax.ShapeDtypeStruct((Bo, L, N_EMBD), jnp.float32)
  p0 = seq0 // G
  common = dict(
      grid=(nseq // G,),
      out_specs=pl.BlockSpec((G, L, N_EMBD), lambda i: (i + p0, 0, 0)),
      out_shape=out_shape,
      compiler_params=pltpu.CompilerParams(
          dimension_semantics=("parallel",)),
  )
  h_spec = pl.BlockSpec((G * L, R), lambda i: (i, 0))
  w_spec = pl.BlockSpec((N_EMBD, R), lambda i: (0, 0))
  if y_prev is None:
    return pl.pallas_call(
        functools.partial(_proj_body, L, G),
        in_specs=[h_spec, w_spec],
        **common,
    )(h, w)
  return pl.pallas_call(
      functools.partial(_proj_body_alias, L, G),
      in_specs=[pl.BlockSpec(memory_space=pl.ANY), h_spec, w_spec],
      input_output_aliases={0: 0},
      **common,
  )(y_prev, h, w)


N_PHASES = 4


def kernel(x, embed_in_weight, embed_out_weight):
  Bo, L = x.shape
  xi = x.astype(jnp.int32)
  nseq = Bo // N_PHASES
  gather = _make_gather(nseq * L)
  hs = [gather(embed_in_weight, xi[p * nseq:(p + 1) * nseq].reshape(-1))
        for p in range(N_PHASES)]
  y = None
  for p in range(N_PHASES):
    y = _project_phase(y, hs[p], embed_out_weight, Bo, L,
                       seq0=p * nseq, nseq=nseq, G=64)
  return y
